# baseline (speedup 1.0000x reference)
import functools

import jax
import jax.numpy as jnp
from jax import lax
from jax.experimental import pallas as pl
from jax.experimental.pallas import tpu as pltpu
from jax.experimental.pallas import tpu_sc as plsc

N = 50000
E = 800000
CH = 64
HALF = 32
NS = 16
CHUNK = 128
TRASH = N
E_PAD = 819200
NCH_PT = E_PAD // (NS * CHUNK)
GB = 2
NG5 = NCH_PT
UNROLL = 8
DGB = 8
N_SC = 51200
ROWS_PT = N_SC // NS
ZROWS = 128
ROWS_BLK = 2000
NBLK = N // ROWS_BLK

@functools.cache
def _mesh():
    return plsc.VectorSubcoreMesh(core_axis_name="c", subcore_axis_name="s",
                                  num_cores=2, num_subcores=NS)
_SC_PARAMS = pltpu.CompilerParams(use_tc_tiling_on_sc=False)


def _full16(v, dtype=jnp.float32):
    return jnp.full((16,), v, dtype)


def _deg_body(cols_hbm, rows_hbm, degf_hbm, degb_hbm,
              idxv, onesb, zbuf, acc, sem_i, sem_s):
    c = lax.axis_index("c")
    s = lax.axis_index("s")
    tbase = s * (NCH_PT * CHUNK)

    @pl.loop(0, ZROWS)
    def _(i):
        zbuf[i] = _full16(0.0)

    @pl.loop(0, CHUNK)
    def _(i):
        onesb[i] = _full16(1.0)

    r0 = s * ROWS_PT

    @pl.loop(0, ROWS_PT // ZROWS)
    def _(zi):
        pltpu.sync_copy(zbuf, acc.at[pl.ds(r0 + zi * ZROWS, ZROWS)])

    plsc.subcore_barrier()

    @pl.loop(0, NCH_PT // DGB)
    def _(k):
        for j in range(DGB):
            base = tbase + (k * DGB + j) * CHUNK

            @pl.when(c == 0)
            def _():
                pltpu.async_copy(cols_hbm.at[pl.ds(base, CHUNK)],
                                 idxv.at[j], sem_i)

            @pl.when(c == 1)
            def _():
                pltpu.async_copy(rows_hbm.at[pl.ds(base, CHUNK)],
                                 idxv.at[j], sem_i)

        for j in range(DGB):
            pltpu.make_async_copy(cols_hbm.at[pl.ds(0, CHUNK)],
                                  idxv.at[j], sem_i).wait()
        for j in range(DGB):
            pltpu.async_copy(onesb, acc.at[idxv.at[j]], sem_s, add=True)
        for j in range(DGB):
            pltpu.make_async_copy(onesb, acc.at[idxv.at[j]], sem_s).wait()

    plsc.subcore_barrier()

    @pl.when(c == 0)
    def _():
        pltpu.sync_copy(acc.at[pl.ds(r0, ROWS_PT)], degf_hbm.at[pl.ds(r0, ROWS_PT)])

    @pl.when(c == 1)
    def _():
        pltpu.sync_copy(acc.at[pl.ds(r0, ROWS_PT)], degb_hbm.at[pl.ds(r0, ROWS_PT)])


def _sc_degrees(col_s, row_s):
    f = pl.kernel(
        _deg_body,
        out_type=[
            jax.ShapeDtypeStruct((N_SC, 16), jnp.float32),
            jax.ShapeDtypeStruct((N_SC, 16), jnp.float32),
        ],
        mesh=_mesh(),
        scratch_types=[
            pltpu.VMEM((DGB, CHUNK), jnp.int32),
            pltpu.VMEM((CHUNK, 16), jnp.float32),
            pltpu.VMEM((ZROWS, 16), jnp.float32),
            pltpu.VMEM_SHARED((N_SC, 16), jnp.float32),
            pltpu.SemaphoreType.DMA,
            pltpu.SemaphoreType.DMA,
        ],
        compiler_params=_SC_PARAMS,
    )
    return f(col_s, row_s)


def _seg_sum_pipeline(c, tbase, u0_hbm, u1_hbm, g0_hbm, s0_hbm,
                      g1_hbm, s1_hbm, rowv, colv, buf, acc, sem_i, sem_g,
                      sem_s):
    def idx_start(p, g):
        base = tbase + g * CHUNK
        s8 = p % 8
        sem = sem_i[p % 2]

        @pl.when(c == 0)
        def _():
            pltpu.async_copy(g0_hbm.at[pl.ds(base, CHUNK)], rowv.at[s8], sem)
            pltpu.async_copy(s0_hbm.at[pl.ds(base, CHUNK)], colv.at[s8], sem)

        @pl.when(c == 1)
        def _():
            pltpu.async_copy(g1_hbm.at[pl.ds(base, CHUNK)], rowv.at[s8], sem)
            pltpu.async_copy(s1_hbm.at[pl.ds(base, CHUNK)], colv.at[s8], sem)

    def idx_wait(p):
        s8 = p % 8
        sem = sem_i[p % 2]
        pltpu.make_async_copy(g0_hbm.at[pl.ds(0, CHUNK)],
                              rowv.at[s8], sem).wait()
        pltpu.make_async_copy(g0_hbm.at[pl.ds(0, CHUNK)],
                              colv.at[s8], sem).wait()

    def gather_start(p):
        s4, s8 = p % 4, p % 8

        @pl.when(c == 0)
        def _():
            pltpu.async_copy(u0_hbm.at[rowv.at[s8]], buf.at[s4], sem_g[s4])

        @pl.when(c == 1)
        def _():
            pltpu.async_copy(u1_hbm.at[rowv.at[s8]], buf.at[s4], sem_g[s4])

    def gather_wait(p):
        s4, s8 = p % 4, p % 8
        pltpu.make_async_copy(u0_hbm.at[rowv.at[s8]], buf.at[s4],
                              sem_g[s4]).wait()

    def scatter_start(p):
        s4, s8 = p % 4, p % 8
        pltpu.async_copy(buf.at[s4], acc.at[colv.at[s8]], sem_s[s4],
                         add=True)

    def scatter_wait(p):
        s4, s8 = p % 4, p % 8
        pltpu.make_async_copy(buf.at[s4], acc.at[colv.at[s8]],
                              sem_s[s4]).wait()

    idx_start(0, 0)
    idx_start(1, 1)

    @pl.loop(0, NG5 // UNROLL)
    def _(k):
        for p in range(UNROLL):
            g = k * UNROLL + p

            if p >= 4:
                scatter_wait(p - 4)
            else:
                @pl.when(k > 0)
                def _():
                    scatter_wait(p + 4)

            idx_wait(p)
            gather_start(p)

            if p >= 3:
                gather_wait(p - 3)
                scatter_start(p - 3)
            else:
                @pl.when(k > 0)
                def _():
                    gather_wait(p + 5)
                    scatter_start(p + 5)

            if p < 6:
                idx_start(p + 2, g + 2)
            else:
                @pl.when(k + 1 < NG5 // UNROLL)
                def _():
                    idx_start(p + 2, g + 2)

    for p in (5, 6, 7):
        gather_wait(p)
        scatter_start(p)
    for p in (4, 5, 6, 7):
        scatter_wait(p)


def _make_edge_body_v5(width):
    def body(u0_hbm, u1_hbm, g0_hbm, s0_hbm, g1_hbm, s1_hbm,
             out0_hbm, out1_hbm, rowv, colv, buf, zbuf, acc,
             sem_i0, sem_i1, sem_g0, sem_g1, sem_g2, sem_g3,
             sem_s0, sem_s1, sem_s2, sem_s3):
        c = lax.axis_index("c")
        s = lax.axis_index("s")
        tbase = s * (NCH_PT * CHUNK)
        r0 = s * ROWS_PT

        @pl.loop(0, ZROWS)
        def _(i):
            for w in range(width // 16):
                zbuf[i, pl.ds(w * 16, 16)] = _full16(0.0)

        @pl.loop(0, ROWS_PT // ZROWS)
        def _(zi):
            pltpu.sync_copy(zbuf, acc.at[pl.ds(r0 + zi * ZROWS, ZROWS)])

        plsc.subcore_barrier()

        _seg_sum_pipeline(c, tbase, u0_hbm, u1_hbm, g0_hbm, s0_hbm,
                          g1_hbm, s1_hbm, rowv, colv, buf, acc,
                          [sem_i0, sem_i1], [sem_g0, sem_g1, sem_g2, sem_g3],
                          [sem_s0, sem_s1, sem_s2, sem_s3])

        plsc.subcore_barrier()

        @pl.when(c == 0)
        def _():
            pltpu.sync_copy(acc.at[pl.ds(r0, ROWS_PT)],
                            out0_hbm.at[pl.ds(r0, ROWS_PT)])

        @pl.when(c == 1)
        def _():
            pltpu.sync_copy(acc.at[pl.ds(r0, ROWS_PT)],
                            out1_hbm.at[pl.ds(r0, ROWS_PT)])

    return body


def _make_pair_body(width):
    def body(ui0_hbm, ui1_hbm, uo0_hbm, uo1_hbm,
             rg_hbm, cs_hbm, cg_hbm, rs_hbm,
             ai0_hbm, ai1_hbm, ao0_hbm, ao1_hbm,
             rowv, colv, buf, zbuf, acc,
             sem_i0, sem_i1, sem_g0, sem_g1, sem_g2, sem_g3,
             sem_s0, sem_s1, sem_s2, sem_s3):
        c = lax.axis_index("c")
        s = lax.axis_index("s")
        tbase = s * (NCH_PT * CHUNK)
        r0 = s * ROWS_PT
        sem_i = [sem_i0, sem_i1]
        sem_g = [sem_g0, sem_g1, sem_g2, sem_g3]
        sem_s = [sem_s0, sem_s1, sem_s2, sem_s3]

        def zero_stripe():
            @pl.loop(0, ROWS_PT // ZROWS)
            def _(zi):
                pltpu.sync_copy(zbuf, acc.at[pl.ds(r0 + zi * ZROWS, ZROWS)])

        def copy_out(o0, o1):
            @pl.when(c == 0)
            def _():
                pltpu.sync_copy(acc.at[pl.ds(r0, ROWS_PT)],
                                o0.at[pl.ds(r0, ROWS_PT)])

            @pl.when(c == 1)
            def _():
                pltpu.sync_copy(acc.at[pl.ds(r0, ROWS_PT)],
                                o1.at[pl.ds(r0, ROWS_PT)])

        @pl.loop(0, ZROWS)
        def _(i):
            for w in range(width // 16):
                zbuf[i, pl.ds(w * 16, 16)] = _full16(0.0)

        zero_stripe()
        plsc.subcore_barrier()
        _seg_sum_pipeline(c, tbase, ui0_hbm, ui1_hbm, rg_hbm, cs_hbm,
                          rg_hbm, cs_hbm, rowv, colv, buf, acc,
                          sem_i, sem_g, sem_s)
        plsc.subcore_barrier()
        copy_out(ai0_hbm, ai1_hbm)
        zero_stripe()
        plsc.subcore_barrier()
        _seg_sum_pipeline(c, tbase, uo0_hbm, uo1_hbm, cg_hbm, rs_hbm,
                          cg_hbm, rs_hbm, rowv, colv, buf, acc,
                          sem_i, sem_g, sem_s)
        plsc.subcore_barrier()
        copy_out(ao0_hbm, ao1_hbm)

    return body


def _sc_pair_pass(ui0, ui1, uo0, uo1, row_g, col_s, col_g, row_s):
    f = pl.kernel(
        _make_pair_body(HALF),
        out_type=[jax.ShapeDtypeStruct((N_SC, HALF), jnp.float32)] * 4,
        mesh=_mesh(),
        scratch_types=[
            pltpu.VMEM((8, CHUNK), jnp.int32),
            pltpu.VMEM((8, CHUNK), jnp.int32),
            pltpu.VMEM((4, CHUNK, HALF), jnp.float32),
            pltpu.VMEM((ZROWS, HALF), jnp.float32),
            pltpu.VMEM_SHARED((N_SC, HALF), jnp.float32),
        ] + [pltpu.SemaphoreType.DMA] * 10,
        compiler_params=_SC_PARAMS,
    )
    return f(ui0, ui1, uo0, uo1, row_g, col_s, col_g, row_s)


def _sc_edge_pass(u0, u1, g0, s0, g1, s1, width):
    f = pl.kernel(
        _make_edge_body_v5(width),
        out_type=[
            jax.ShapeDtypeStruct((N_SC, width), jnp.float32),
            jax.ShapeDtypeStruct((N_SC, width), jnp.float32),
        ],
        mesh=_mesh(),
        scratch_types=[
            pltpu.VMEM((8, CHUNK), jnp.int32),
            pltpu.VMEM((8, CHUNK), jnp.int32),
            pltpu.VMEM((4, CHUNK, width), jnp.float32),
            pltpu.VMEM((ZROWS, width), jnp.float32),
            pltpu.VMEM_SHARED((N_SC, width), jnp.float32),
        ] + [pltpu.SemaphoreType.DMA] * 10,
        compiler_params=_SC_PARAMS,
    )
    return f(u0, u1, g0, s0, g1, s1)


def _row_spec(cols):
    return pl.BlockSpec((ROWS_BLK, cols), lambda i: (i, 0))


def _rep_spec(r, cols):
    return pl.BlockSpec((r, cols), lambda i: (0, 0))


def _embed_pre(x, embed, wi, bi, wci, wo, bo, wco, dinv_f, dinv_b):
    def body(x_ref, emb_ref, wi_ref, bi_ref, wci_ref, wo_ref, bo_ref,
             wco_ref, df_ref, db_ref, h_ref,
             xwi_ref, ui0_ref, ui1_ref, xwo_ref, uo0_ref, uo1_ref):
        ids = x_ref[...]
        onehot = (ids == lax.broadcasted_iota(jnp.int32, (ROWS_BLK, 32), 1)
                  ).astype(jnp.float32)
        h = jnp.dot(onehot, emb_ref[...], preferred_element_type=jnp.float32)
        h_ref[...] = h
        xp = jnp.maximum(
            jnp.dot(h, wi_ref[...], preferred_element_type=jnp.float32)
            + bi_ref[...], 0.0)
        xw = jnp.dot(xp, wci_ref[...], preferred_element_type=jnp.float32)
        xwi_ref[...] = xw
        u = xw * df_ref[...]
        ui0_ref[...] = u[:, :HALF]
        ui1_ref[...] = u[:, HALF:]
        xp = jnp.maximum(
            jnp.dot(h, wo_ref[...], preferred_element_type=jnp.float32)
            + bo_ref[...], 0.0)
        xw = jnp.dot(xp, wco_ref[...], preferred_element_type=jnp.float32)
        xwo_ref[...] = xw
        u = xw * db_ref[...]
        uo0_ref[...] = u[:, :HALF]
        uo1_ref[...] = u[:, HALF:]

    return pl.pallas_call(
        body,
        grid=(NBLK,),
        in_specs=[
            _row_spec(1),
            _rep_spec(32, CH),
            _rep_spec(CH, CH), _rep_spec(1, CH), _rep_spec(CH, CH),
            _rep_spec(CH, CH), _rep_spec(1, CH), _rep_spec(CH, CH),
            _row_spec(1), _row_spec(1),
        ],
        out_specs=[
            _row_spec(CH),
            _row_spec(CH), _row_spec(HALF), _row_spec(HALF),
            _row_spec(CH), _row_spec(HALF), _row_spec(HALF),
        ],
        out_shape=[
            jax.ShapeDtypeStruct((N, CH), jnp.float32),
            jax.ShapeDtypeStruct((N, CH), jnp.float32),
            jax.ShapeDtypeStruct((N, HALF), jnp.float32),
            jax.ShapeDtypeStruct((N, HALF), jnp.float32),
            jax.ShapeDtypeStruct((N, CH), jnp.float32),
            jax.ShapeDtypeStruct((N, HALF), jnp.float32),
            jax.ShapeDtypeStruct((N, HALF), jnp.float32),
        ],
    )(x.reshape(N, 1), embed, wi, bi.reshape(1, CH), wci,
      wo, bo.reshape(1, CH), wco, dinv_f, dinv_b)


def _piece_spec(v):
    return _rep_spec(1, CH) if v.shape[0] == 1 else _row_spec(CH)


def _branch_pre(xs, w_pieces, b, wc, dinv):
    n = len(xs)

    def body(*refs):
        xrefs = refs[:n]
        wrefs = refs[n:2 * n]
        b_ref, wc_ref, d_ref, xw_ref, u0_ref, u1_ref = refs[2 * n:]
        a = b_ref[...].astype(jnp.float32)
        for k in range(n):
            a = a + jnp.dot(xrefs[k][...], wrefs[k][...],
                            preferred_element_type=jnp.float32)
        xp = jnp.maximum(a, 0.0)
        xw = jnp.dot(xp, wc_ref[...], preferred_element_type=jnp.float32)
        xw_ref[...] = xw
        u = xw * d_ref[...]
        u0_ref[...] = u[:, :HALF]
        u1_ref[...] = u[:, HALF:]

    return pl.pallas_call(
        body,
        grid=(NBLK,),
        in_specs=[_piece_spec(v) for v in xs] + [_rep_spec(CH, CH)] * n
        + [_rep_spec(1, CH), _rep_spec(CH, CH), _row_spec(1)],
        out_specs=[_row_spec(CH), _row_spec(HALF), _row_spec(HALF)],
        out_shape=[
            jax.ShapeDtypeStruct((N, CH), jnp.float32),
            jax.ShapeDtypeStruct((N, HALF), jnp.float32),
            jax.ShapeDtypeStruct((N, HALF), jnp.float32),
        ],
    )(*xs, *w_pieces, b.reshape(1, CH), wc, dinv)


def _branch_post(a0, a1, xw, dinv, inv_deg, bc):
    def body(a0_ref, a1_ref, xw_ref, d_ref, id_ref, b_ref, o_ref):
        acc = jnp.concatenate([a0_ref[...], a1_ref[...]], axis=1)
        o_ref[...] = jnp.maximum(
            d_ref[...] * acc + xw_ref[...] * id_ref[...] + b_ref[...], 0.0)

    return pl.pallas_call(
        body,
        grid=(NBLK,),
        in_specs=[
            _row_spec(HALF), _row_spec(HALF), _row_spec(CH),
            _row_spec(1), _row_spec(1), _rep_spec(1, CH),
        ],
        out_specs=_row_spec(CH),
        out_shape=jax.ShapeDtypeStruct((N, CH), jnp.float32),
    )(a0[:N], a1[:N], xw, dinv, inv_deg, bc.reshape(1, CH))


def _dinv_post(cnt_f, cnt_b):
    def body(cf_ref, cb_ref, df_ref, db_ref, idf_ref, idb_ref,
             tf_ref, tb_ref):
        deg_f = cf_ref[:, 0:1] + 1.0
        deg_b = cb_ref[:, 0:1] + 1.0
        df = lax.rsqrt(deg_f)
        db = lax.rsqrt(deg_b)
        df_ref[...] = df
        db_ref[...] = db
        idf_ref[...] = 1.0 / deg_f
        idb_ref[...] = 1.0 / deg_b
        tf_ref[...] = jnp.broadcast_to(df, (ROWS_BLK, 16))
        tb_ref[...] = jnp.broadcast_to(db, (ROWS_BLK, 16))

    return pl.pallas_call(
        body,
        grid=(NBLK,),
        in_specs=[_row_spec(16), _row_spec(16)],
        out_specs=[_row_spec(1), _row_spec(1), _row_spec(1), _row_spec(1),
                   _row_spec(16), _row_spec(16)],
        out_shape=[jax.ShapeDtypeStruct((N, 1), jnp.float32)] * 4
        + [jax.ShapeDtypeStruct((N, 16), jnp.float32)] * 2,
    )(cnt_f[:N], cnt_b[:N])


def _col_mean(xs):
    n = len(xs)

    def body(*refs):
        o_ref = refs[n]
        i = pl.program_id(0)

        @pl.when(i == 0)
        def _():
            o_ref[...] = jnp.zeros_like(o_ref)

        for k in range(n):
            o_ref[0:1, k * CH:(k + 1) * CH] += jnp.sum(
                refs[k][...], axis=0, keepdims=True) * (1.0 / N)

    return pl.pallas_call(
        body,
        grid=(NBLK,),
        in_specs=[_row_spec(CH)] * n,
        out_specs=pl.BlockSpec((1, n * CH), lambda i: (0, 0)),
        out_shape=jax.ShapeDtypeStruct((1, n * CH), jnp.float32),
    )(*xs)


def _head(hm, p, noise):
    def body(hm_ref, wh_ref, bh_ref, wm_ref, bm_ref, wv_ref, bv_ref,
             nz_ref, wdi_ref, bdi_ref, wdh_ref, bdh_ref,
             wi0_ref, bi0_ref, wci0_ref, wo0_ref, bo0_ref, wco0_ref,
             mean_ref, var_ref, d2_ref, xwi_ref, xwo_ref):
        h = jnp.maximum(
            jnp.dot(hm_ref[...], wh_ref[...],
                    preferred_element_type=jnp.float32) + bh_ref[...], 0.0)
        mean = 2.0 * jnp.tanh(
            jnp.dot(h, wm_ref[...], preferred_element_type=jnp.float32)
            + bm_ref[...])
        var = 2.0 * jax.nn.sigmoid(
            jnp.dot(h, wv_ref[...], preferred_element_type=jnp.float32)
            + bv_ref[...])
        mean_ref[...] = mean
        var_ref[...] = var
        z = mean + nz_ref[...] * jnp.sqrt(var)
        d = jnp.maximum(
            jnp.dot(z, wdi_ref[...], preferred_element_type=jnp.float32)
            + bdi_ref[...], 0.0)
        d = jnp.maximum(
            jnp.dot(d, wdh_ref[...], preferred_element_type=jnp.float32)
            + bdh_ref[...], 0.0)
        d2_ref[...] = d
        xp = jnp.maximum(
            jnp.dot(d, wi0_ref[...], preferred_element_type=jnp.float32)
            + bi0_ref[...], 0.0)
        xwi_ref[...] = jnp.dot(xp, wci0_ref[...],
                               preferred_element_type=jnp.float32)
        xp = jnp.maximum(
            jnp.dot(d, wo0_ref[...], preferred_element_type=jnp.float32)
            + bo0_ref[...], 0.0)
        xwo_ref[...] = jnp.dot(xp, wco0_ref[...],
                               preferred_element_type=jnp.float32)

    d0 = p["dec_dense"][0]
    ins = [hm,
           p["enc_hidden"]["W"], p["enc_hidden"]["b"].reshape(1, -1),
           p["enc_mean"]["W"], p["enc_mean"]["b"].reshape(1, -1),
           p["enc_var"]["W"], p["enc_var"]["b"].reshape(1, -1),
           noise,
           p["dec_input"]["W"], p["dec_input"]["b"].reshape(1, -1),
           p["dec_hidden"]["W"], p["dec_hidden"]["b"].reshape(1, -1),
           d0["lin_in"]["W"], d0["lin_in"]["b"].reshape(1, -1),
           d0["conv_in"]["W"],
           d0["lin_out"]["W"], d0["lin_out"]["b"].reshape(1, -1),
           d0["conv_out"]["W"]]
    return pl.pallas_call(
        body,
        grid=(1,),
        in_specs=[pl.BlockSpec(v.shape, lambda i: (0, 0)) for v in ins],
        out_specs=[pl.BlockSpec((1, CH), lambda i: (0, 0))] * 5,
        out_shape=[jax.ShapeDtypeStruct((1, CH), jnp.float32)] * 5,
    )(*ins)


def _dec0_outer(sf, sb, dinv_f, inv_deg_f, dinv_b, inv_deg_b,
                xwi, xwo, bi, bo):
    def body(sf_ref, sb_ref, df_ref, idf_ref, db_ref, idb_ref,
             xwi_ref, xwo_ref, bi_ref, bo_ref, xi_ref, xo_ref):
        cf = df_ref[...] * sf_ref[:, 0:1] + idf_ref[...]
        xi_ref[...] = jnp.maximum(cf * xwi_ref[...] + bi_ref[...], 0.0)
        cb = db_ref[...] * sb_ref[:, 1:2] + idb_ref[...]
        xo_ref[...] = jnp.maximum(cb * xwo_ref[...] + bo_ref[...], 0.0)

    return pl.pallas_call(
        body,
        grid=(NBLK,),
        in_specs=[_row_spec(16), _row_spec(16),
                  _row_spec(1), _row_spec(1), _row_spec(1), _row_spec(1),
                  _rep_spec(1, CH), _rep_spec(1, CH),
                  _rep_spec(1, CH), _rep_spec(1, CH)],
        out_specs=[_row_spec(CH), _row_spec(CH)],
        out_shape=[jax.ShapeDtypeStruct((N, CH), jnp.float32)] * 2,
    )(sf[:N], sb[:N], dinv_f, inv_deg_f, dinv_b, inv_deg_b,
      xwi, xwo, bi.reshape(1, CH), bo.reshape(1, CH))


def _out_proj(xs, w, b):
    n = len(xs)
    fo = w.shape[1]

    def body(*refs):
        wrefs = refs[n:2 * n]
        b_ref, o_ref = refs[2 * n:]
        a = jnp.broadcast_to(b_ref[...], o_ref.shape).astype(jnp.float32)
        for k in range(n):
            a = a + jnp.dot(refs[k][...], wrefs[k][...],
                            preferred_element_type=jnp.float32)
        o_ref[...] = a

    def piece_spec(v):
        return _rep_spec(1, CH) if v.shape[0] == 1 else _row_spec(CH)

    w_pieces = [w[k * CH:(k + 1) * CH] for k in range(n)]
    return pl.pallas_call(
        body,
        grid=(NBLK,),
        in_specs=[piece_spec(v) for v in xs] + [_rep_spec(CH, fo)] * n
        + [_rep_spec(1, fo)],
        out_specs=_row_spec(fo),
        out_shape=jax.ShapeDtypeStruct((N, fo), jnp.float32),
    )(*xs, *w_pieces, b.reshape(1, fo))


def _gcn_pair(xs, p, row_g, col_s, col_g, row_s,
              dinv_f, inv_deg_f, dinv_b, inv_deg_b):
    nin = len(xs)
    wi = [p["lin_in"]["W"][k * CH:(k + 1) * CH] for k in range(nin)]
    wo = [p["lin_out"]["W"][k * CH:(k + 1) * CH] for k in range(nin)]
    xwi, ui0, ui1 = _branch_pre(xs, wi, p["lin_in"]["b"],
                                p["conv_in"]["W"], dinv_f)
    xwo, uo0, uo1 = _branch_pre(xs, wo, p["lin_out"]["b"],
                                p["conv_out"]["W"], dinv_b)
    ai0, ai1, ao0, ao1 = _sc_pair_pass(ui0, ui1, uo0, uo1,
                                       row_g, col_s, col_g, row_s)
    xi = _branch_post(ai0, ai1, xwi, dinv_f, inv_deg_f, p["conv_in"]["b"])
    xo = _branch_post(ao0, ao1, xwo, dinv_b, inv_deg_b, p["conv_out"]["b"])
    return xi, xo


def kernel(x, edge_index, params):
    row = edge_index[0]
    col = edge_index[1]
    padz = jnp.zeros((E_PAD - E,), jnp.int32)
    padt = jnp.full((E_PAD - E,), TRASH, jnp.int32)
    row_g = jnp.concatenate([row, padz])
    col_g = jnp.concatenate([col, padz])
    row_s = jnp.concatenate([row, padt])
    col_s = jnp.concatenate([col, padt])
    cnt_f, cnt_b = _sc_degrees(col_s, row_s)
    dinv_f, dinv_b, inv_deg_f, inv_deg_b, dtab_f, dtab_b = _dinv_post(
        cnt_f, cnt_b)

    sf, sb = _sc_edge_pass(dtab_f, dtab_b, row_g, col_s, col_g, row_s, 16)

    h, xwi, ui0, ui1, xwo, uo0, uo1 = _embed_pre(
        x, params["embed"],
        params["enc_dense"][0]["lin_in"]["W"],
        params["enc_dense"][0]["lin_in"]["b"],
        params["enc_dense"][0]["conv_in"]["W"],
        params["enc_dense"][0]["lin_out"]["W"],
        params["enc_dense"][0]["lin_out"]["b"],
        params["enc_dense"][0]["conv_out"]["W"],
        dinv_f, dinv_b)
    ai0, ai1, ao0, ao1 = _sc_pair_pass(ui0, ui1, uo0, uo1,
                                       row_g, col_s, col_g, row_s)
    xi = _branch_post(ai0, ai1, xwi, dinv_f, inv_deg_f,
                      params["enc_dense"][0]["conv_in"]["b"])
    xo = _branch_post(ao0, ao1, xwo, dinv_b, inv_deg_b,
                      params["enc_dense"][0]["conv_out"]["b"])
    xs = [h, xi, xo]
    xi2, xo2 = _gcn_pair(xs, params["enc_dense"][1], row_g, col_s, col_g,
                         row_s, dinv_f, inv_deg_f, dinv_b, inv_deg_b)
    xs = xs + [xi2, xo2]

    hm = _col_mean(xs)
    noise = jax.random.normal(jax.random.key(42), (1, CH), jnp.float32)
    mean, var, d2, xwi0, xwo0 = _head(hm, params, noise)

    d0 = params["dec_dense"][0]
    dxi, dxo = _dec0_outer(sf, sb, dinv_f, inv_deg_f, dinv_b, inv_deg_b,
                           xwi0, xwo0, d0["conv_in"]["b"],
                           d0["conv_out"]["b"])
    dxs = [d2, dxi, dxo]
    dxi2, dxo2 = _gcn_pair(dxs, params["dec_dense"][1], row_g, col_s, col_g,
                           row_s, dinv_f, inv_deg_f, dinv_b, inv_deg_b)
    dxs = dxs + [dxi2, dxo2]

    y = _out_proj(dxs, params["dec_output"]["W"], params["dec_output"]["b"])
    return (mean.reshape(CH), var.reshape(CH), y)


# speedup vs baseline: 1.1242x; 1.1242x over previous
import functools

import jax
import jax.numpy as jnp
from jax import lax
from jax.experimental import pallas as pl
from jax.experimental.pallas import tpu as pltpu
from jax.experimental.pallas import tpu_sc as plsc

N = 50000
E = 800000
CH = 64
HALF = 32
NS = 16
CHUNK = 128
TRASH = N
E_PAD = 819200
NCH_PT = E_PAD // (NS * CHUNK)
GB = 2
NG5 = NCH_PT
UNROLL = 8
DGB = 8
N_SC = 51200
ROWS_PT = N_SC // NS
ZROWS = 128
ROWS_BLK = 2000
NBLK = N // ROWS_BLK

@functools.cache
def _mesh():
    return plsc.VectorSubcoreMesh(core_axis_name="c", subcore_axis_name="s",
                                  num_cores=2, num_subcores=NS)
_SC_PARAMS = pltpu.CompilerParams(use_tc_tiling_on_sc=False)
_SC_PARAMS_REG = pltpu.CompilerParams(use_tc_tiling_on_sc=False,
                                      needs_layout_passes=False)


def _full16(v, dtype=jnp.float32):
    return jnp.full((16,), v, dtype)


HROWS = N_SC // 16
HSTRIPE = HROWS // NS
MROWS = 128
IDXB = 1024


def _reg_hist_zero(hist):
    @pl.loop(0, HROWS)
    def _(i):
        hist[i] = _full16(0.0)


def _reg_hist_merge(hist, iotav, acc, sem):
    @pl.loop(0, HROWS // MROWS)
    def _(m):
        @pl.loop(0, MROWS // 16)
        def _(i):
            iotav[pl.ds(i * 16, 16)] = (lax.iota(jnp.int32, 16)
                                        + (m * MROWS + i * 16))
        pltpu.async_copy(hist.at[pl.ds(m * MROWS, MROWS)],
                         acc.at[iotav], sem, add=True).wait()


def _deg_body_reg(cols_hbm, rows_hbm, degf_hbm, degb_hbm,
                  idxv, hist, iotav, acc, zbuf, sem_i, sem_m):
    c = lax.axis_index("c")
    s = lax.axis_index("s")
    ebase = s * (NCH_PT * CHUNK)
    EPT = NCH_PT * CHUNK

    _reg_hist_zero(hist)

    @pl.loop(0, HSTRIPE // 2)
    def _(i):
        zbuf[i] = _full16(0.0)

    r0 = s * HSTRIPE

    @pl.loop(0, 2)
    def _(zi):
        pltpu.sync_copy(zbuf, acc.at[pl.ds(r0 + zi * (HSTRIPE // 2),
                                           HSTRIPE // 2)])

    ones = _full16(1.0)

    def load(slot, b):
        @pl.when(c == 0)
        def _():
            pltpu.async_copy(cols_hbm.at[pl.ds(ebase + b * IDXB, IDXB)],
                             idxv.at[slot], sem_i)

        @pl.when(c == 1)
        def _():
            pltpu.async_copy(rows_hbm.at[pl.ds(ebase + b * IDXB, IDXB)],
                             idxv.at[slot], sem_i)

    def wait(slot):
        pltpu.make_async_copy(cols_hbm.at[pl.ds(0, IDXB)],
                              idxv.at[slot], sem_i).wait()

    def consume(slot):
        @pl.loop(0, IDXB // 16)
        def _(i):
            v = idxv[slot, pl.ds(i * 16, 16)]
            r = jax.lax.shift_right_logical(v, 4)
            col = jax.lax.bitwise_and(v, 15)
            plsc.addupdate_scatter(hist, [r, col], ones)

    NB = EPT // IDXB
    load(0, 0)

    @pl.loop(0, NB // 2)
    def _(b):
        wait(0)

        @pl.when(2 * b + 1 < NB)
        def _():
            load(1, 2 * b + 1)
        consume(0)

        wait(1)

        @pl.when(2 * b + 2 < NB)
        def _():
            load(0, 2 * b + 2)
        consume(1)

    plsc.subcore_barrier()
    _reg_hist_merge(hist, iotav, acc, sem_m)
    plsc.subcore_barrier()

    @pl.when(c == 0)
    def _():
        pltpu.sync_copy(acc.at[pl.ds(r0, HSTRIPE)],
                        degf_hbm.at[pl.ds(r0, HSTRIPE)])

    @pl.when(c == 1)
    def _():
        pltpu.sync_copy(acc.at[pl.ds(r0, HSTRIPE)],
                        degb_hbm.at[pl.ds(r0, HSTRIPE)])


def _sc_degrees(col_s, row_s):
    f = pl.kernel(
        _deg_body_reg,
        out_type=[
            jax.ShapeDtypeStruct((HROWS, 16), jnp.float32),
            jax.ShapeDtypeStruct((HROWS, 16), jnp.float32),
        ],
        mesh=_mesh(),
        scratch_types=[
            pltpu.VMEM((2, IDXB), jnp.int32),
            pltpu.VMEM((HROWS, 16), jnp.float32),
            pltpu.VMEM((MROWS,), jnp.int32),
            pltpu.VMEM_SHARED((HROWS, 16), jnp.float32),
            pltpu.VMEM((HSTRIPE // 2, 16), jnp.float32),
            pltpu.SemaphoreType.DMA,
            pltpu.SemaphoreType.DMA,
        ],
        compiler_params=_SC_PARAMS_REG,
    )
    return f(col_s, row_s)


def _coef_body_reg(tf_hbm, tb_hbm, g0_hbm, s0_hbm, g1_hbm, s1_hbm,
                   sf_hbm, sb_hbm, gidxv, sidxv, dtab, hist, iotav, acc,
                   zbuf, sem_i, sem_m):
    c = lax.axis_index("c")
    s = lax.axis_index("s")
    ebase = s * (NCH_PT * CHUNK)
    EPT = NCH_PT * CHUNK

    _reg_hist_zero(hist)

    @pl.loop(0, HSTRIPE // 2)
    def _(i):
        zbuf[i] = _full16(0.0)

    r0 = s * HSTRIPE

    @pl.loop(0, 2)
    def _(zi):
        pltpu.sync_copy(zbuf, acc.at[pl.ds(r0 + zi * (HSTRIPE // 2),
                                           HSTRIPE // 2)])

    @pl.when(c == 0)
    def _():
        pltpu.sync_copy(tf_hbm, dtab)

    @pl.when(c == 1)
    def _():
        pltpu.sync_copy(tb_hbm, dtab)

    def load(slot, b):
        @pl.when(c == 0)
        def _():
            pltpu.async_copy(g0_hbm.at[pl.ds(ebase + b * IDXB, IDXB)],
                             gidxv.at[slot], sem_i)
            pltpu.async_copy(s0_hbm.at[pl.ds(ebase + b * IDXB, IDXB)],
                             sidxv.at[slot], sem_i)

        @pl.when(c == 1)
        def _():
            pltpu.async_copy(g1_hbm.at[pl.ds(ebase + b * IDXB, IDXB)],
                             gidxv.at[slot], sem_i)
            pltpu.async_copy(s1_hbm.at[pl.ds(ebase + b * IDXB, IDXB)],
                             sidxv.at[slot], sem_i)

    def wait(slot):
        pltpu.make_async_copy(g0_hbm.at[pl.ds(0, IDXB)],
                              gidxv.at[slot], sem_i).wait()
        pltpu.make_async_copy(g0_hbm.at[pl.ds(0, IDXB)],
                              sidxv.at[slot], sem_i).wait()

    def consume(slot):
        @pl.loop(0, IDXB // 16)
        def _(i):
            gv = gidxv[slot, pl.ds(i * 16, 16)]
            gr = jax.lax.shift_right_logical(gv, 4)
            gc = jax.lax.bitwise_and(gv, 15)
            vals = plsc.load_gather(dtab, [gr, gc])
            sv = sidxv[slot, pl.ds(i * 16, 16)]
            sr = jax.lax.shift_right_logical(sv, 4)
            sc_ = jax.lax.bitwise_and(sv, 15)
            plsc.addupdate_scatter(hist, [sr, sc_], vals)

    NB = EPT // IDXB
    load(0, 0)

    @pl.loop(0, NB // 2)
    def _(b):
        wait(0)

        @pl.when(2 * b + 1 < NB)
        def _():
            load(1, 2 * b + 1)
        consume(0)

        wait(1)

        @pl.when(2 * b + 2 < NB)
        def _():
            load(0, 2 * b + 2)
        consume(1)

    plsc.subcore_barrier()
    _reg_hist_merge(hist, iotav, acc, sem_m)
    plsc.subcore_barrier()

    @pl.when(c == 0)
    def _():
        pltpu.sync_copy(acc.at[pl.ds(r0, HSTRIPE)],
                        sf_hbm.at[pl.ds(r0, HSTRIPE)])

    @pl.when(c == 1)
    def _():
        pltpu.sync_copy(acc.at[pl.ds(r0, HSTRIPE)],
                        sb_hbm.at[pl.ds(r0, HSTRIPE)])


def _sc_coef(tab_f, tab_b, row_g, col_s, col_g, row_s):
    f = pl.kernel(
        _coef_body_reg,
        out_type=[
            jax.ShapeDtypeStruct((HROWS, 16), jnp.float32),
            jax.ShapeDtypeStruct((HROWS, 16), jnp.float32),
        ],
        mesh=_mesh(),
        scratch_types=[
            pltpu.VMEM((2, IDXB), jnp.int32),
            pltpu.VMEM((2, IDXB), jnp.int32),
            pltpu.VMEM((HROWS, 16), jnp.float32),
            pltpu.VMEM((HROWS, 16), jnp.float32),
            pltpu.VMEM((MROWS,), jnp.int32),
            pltpu.VMEM_SHARED((HROWS, 16), jnp.float32),
            pltpu.VMEM((HSTRIPE // 2, 16), jnp.float32),
            pltpu.SemaphoreType.DMA,
            pltpu.SemaphoreType.DMA,
        ],
        compiler_params=_SC_PARAMS_REG,
    )
    return f(tab_f, tab_b, row_g, col_s, col_g, row_s)


def _make_edge_body_v5(width):
    def body(u0_hbm, u1_hbm, g0_hbm, s0_hbm, g1_hbm, s1_hbm,
             out0_hbm, out1_hbm, rowv, colv, buf, zbuf, acc,
             sem_i0, sem_i1, sem_g0, sem_g1, sem_g2, sem_g3,
             sem_s0, sem_s1, sem_s2, sem_s3):
        sem_i = [sem_i0, sem_i1]
        sem_g = [sem_g0, sem_g1, sem_g2, sem_g3]
        sem_s = [sem_s0, sem_s1, sem_s2, sem_s3]
        c = lax.axis_index("c")
        s = lax.axis_index("s")
        tbase = s * (NCH_PT * CHUNK)

        @pl.loop(0, ZROWS)
        def _(i):
            for w in range(width // 16):
                zbuf[i, pl.ds(w * 16, 16)] = _full16(0.0)

        r0 = s * ROWS_PT

        @pl.loop(0, ROWS_PT // ZROWS)
        def _(zi):
            pltpu.sync_copy(zbuf, acc.at[pl.ds(r0 + zi * ZROWS, ZROWS)])

        plsc.subcore_barrier()

        def idx_start(p, g):
            base = tbase + g * CHUNK
            s8 = p % 8
            sem = sem_i[p % 2]

            @pl.when(c == 0)
            def _():
                pltpu.async_copy(g0_hbm.at[pl.ds(base, CHUNK)],
                                 rowv.at[s8], sem)
                pltpu.async_copy(s0_hbm.at[pl.ds(base, CHUNK)],
                                 colv.at[s8], sem)

            @pl.when(c == 1)
            def _():
                pltpu.async_copy(g1_hbm.at[pl.ds(base, CHUNK)],
                                 rowv.at[s8], sem)
                pltpu.async_copy(s1_hbm.at[pl.ds(base, CHUNK)],
                                 colv.at[s8], sem)

        def idx_wait(p):
            s8 = p % 8
            sem = sem_i[p % 2]
            pltpu.make_async_copy(g0_hbm.at[pl.ds(0, CHUNK)],
                                  rowv.at[s8], sem).wait()
            pltpu.make_async_copy(g0_hbm.at[pl.ds(0, CHUNK)],
                                  colv.at[s8], sem).wait()

        def gather_start(p):
            s4, s8 = p % 4, p % 8

            @pl.when(c == 0)
            def _():
                pltpu.async_copy(u0_hbm.at[rowv.at[s8]], buf.at[s4],
                                 sem_g[s4])

            @pl.when(c == 1)
            def _():
                pltpu.async_copy(u1_hbm.at[rowv.at[s8]], buf.at[s4],
                                 sem_g[s4])

        def gather_wait(p):
            s4, s8 = p % 4, p % 8
            pltpu.make_async_copy(u0_hbm.at[rowv.at[s8]], buf.at[s4],
                                  sem_g[s4]).wait()

        def scatter_start(p):
            s4, s8 = p % 4, p % 8
            pltpu.async_copy(buf.at[s4], acc.at[colv.at[s8]], sem_s[s4],
                             add=True)

        def scatter_wait(p):
            s4, s8 = p % 4, p % 8
            pltpu.make_async_copy(buf.at[s4], acc.at[colv.at[s8]],
                                  sem_s[s4]).wait()

        idx_start(0, 0)
        idx_start(1, 1)

        @pl.loop(0, NG5 // UNROLL)
        def _(k):
            for p in range(UNROLL):
                g = k * UNROLL + p

                if p >= 4:
                    scatter_wait(p - 4)
                else:
                    @pl.when(k > 0)
                    def _():
                        scatter_wait(p + 4)

                idx_wait(p)

                gather_start(p)

                if p >= 3:
                    gather_wait(p - 3)
                    scatter_start(p - 3)
                else:
                    @pl.when(k > 0)
                    def _():
                        gather_wait(p + 5)
                        scatter_start(p + 5)

                if p < 6:
                    idx_start(p + 2, g + 2)
                else:
                    @pl.when(k + 1 < NG5 // UNROLL)
                    def _():
                        idx_start(p + 2, g + 2)

        for p, g in ((5, NG5 - 3), (6, NG5 - 2), (7, NG5 - 1)):
            gather_wait(p)
            scatter_start(p)
        for p in (4, 5, 6, 7):
            scatter_wait(p)

        plsc.subcore_barrier()

        @pl.when(c == 0)
        def _():
            pltpu.sync_copy(acc.at[pl.ds(r0, ROWS_PT)],
                            out0_hbm.at[pl.ds(r0, ROWS_PT)])

        @pl.when(c == 1)
        def _():
            pltpu.sync_copy(acc.at[pl.ds(r0, ROWS_PT)],
                            out1_hbm.at[pl.ds(r0, ROWS_PT)])

    return body


def _sc_edge_pass(u0, u1, g0, s0, g1, s1, width):
    f = pl.kernel(
        _make_edge_body_v5(width),
        out_type=[
            jax.ShapeDtypeStruct((N_SC, width), jnp.float32),
            jax.ShapeDtypeStruct((N_SC, width), jnp.float32),
        ],
        mesh=_mesh(),
        scratch_types=[
            pltpu.VMEM((8, CHUNK), jnp.int32),
            pltpu.VMEM((8, CHUNK), jnp.int32),
            pltpu.VMEM((4, CHUNK, width), jnp.float32),
            pltpu.VMEM((ZROWS, width), jnp.float32),
            pltpu.VMEM_SHARED((N_SC, width), jnp.float32),
        ] + [pltpu.SemaphoreType.DMA] * 10,
        compiler_params=_SC_PARAMS,
    )
    return f(u0, u1, g0, s0, g1, s1)


def _row_spec(cols):
    return pl.BlockSpec((ROWS_BLK, cols), lambda i: (i, 0))


def _rep_spec(r, cols):
    return pl.BlockSpec((r, cols), lambda i: (0, 0))


def _embed_pre(x, embed, wi, bi, wci, wo, bo, wco, dinv_f, dinv_b):
    def body(x_ref, emb_ref, wi_ref, bi_ref, wci_ref, wo_ref, bo_ref,
             wco_ref, df_ref, db_ref, h_ref,
             xwi_ref, ui0_ref, ui1_ref, xwo_ref, uo0_ref, uo1_ref):
        ids = x_ref[...]
        onehot = (ids == lax.broadcasted_iota(jnp.int32, (ROWS_BLK, 32), 1)
                  ).astype(jnp.float32)
        h = jnp.dot(onehot, emb_ref[...], preferred_element_type=jnp.float32)
        h_ref[...] = h
        xp = jnp.maximum(
            jnp.dot(h, wi_ref[...], preferred_element_type=jnp.float32)
            + bi_ref[...], 0.0)
        xw = jnp.dot(xp, wci_ref[...], preferred_element_type=jnp.float32)
        xwi_ref[...] = xw
        u = xw * df_ref[...]
        ui0_ref[...] = u[:, :HALF]
        ui1_ref[...] = u[:, HALF:]
        xp = jnp.maximum(
            jnp.dot(h, wo_ref[...], preferred_element_type=jnp.float32)
            + bo_ref[...], 0.0)
        xw = jnp.dot(xp, wco_ref[...], preferred_element_type=jnp.float32)
        xwo_ref[...] = xw
        u = xw * db_ref[...]
        uo0_ref[...] = u[:, :HALF]
        uo1_ref[...] = u[:, HALF:]

    return pl.pallas_call(
        body,
        grid=(NBLK,),
        in_specs=[
            _row_spec(1),
            _rep_spec(32, CH),
            _rep_spec(CH, CH), _rep_spec(1, CH), _rep_spec(CH, CH),
            _rep_spec(CH, CH), _rep_spec(1, CH), _rep_spec(CH, CH),
            _row_spec(1), _row_spec(1),
        ],
        out_specs=[
            _row_spec(CH),
            _row_spec(CH), _row_spec(HALF), _row_spec(HALF),
            _row_spec(CH), _row_spec(HALF), _row_spec(HALF),
        ],
        out_shape=[
            jax.ShapeDtypeStruct((N, CH), jnp.float32),
            jax.ShapeDtypeStruct((N, CH), jnp.float32),
            jax.ShapeDtypeStruct((N, HALF), jnp.float32),
            jax.ShapeDtypeStruct((N, HALF), jnp.float32),
            jax.ShapeDtypeStruct((N, CH), jnp.float32),
            jax.ShapeDtypeStruct((N, HALF), jnp.float32),
            jax.ShapeDtypeStruct((N, HALF), jnp.float32),
        ],
    )(x.reshape(N, 1), embed, wi, bi.reshape(1, CH), wci,
      wo, bo.reshape(1, CH), wco, dinv_f, dinv_b)


def _piece_spec(v):
    return _rep_spec(1, CH) if v.shape[0] == 1 else _row_spec(CH)


def _branch_pre(xs, w_pieces, b, wc, dinv):
    n = len(xs)

    def body(*refs):
        xrefs = refs[:n]
        wrefs = refs[n:2 * n]
        b_ref, wc_ref, d_ref, xw_ref, u0_ref, u1_ref = refs[2 * n:]
        a = b_ref[...].astype(jnp.float32)
        for k in range(n):
            a = a + jnp.dot(xrefs[k][...], wrefs[k][...],
                            preferred_element_type=jnp.float32)
        xp = jnp.maximum(a, 0.0)
        xw = jnp.dot(xp, wc_ref[...], preferred_element_type=jnp.float32)
        xw_ref[...] = xw
        u = xw * d_ref[...]
        u0_ref[...] = u[:, :HALF]
        u1_ref[...] = u[:, HALF:]

    return pl.pallas_call(
        body,
        grid=(NBLK,),
        in_specs=[_piece_spec(v) for v in xs] + [_rep_spec(CH, CH)] * n
        + [_rep_spec(1, CH), _rep_spec(CH, CH), _row_spec(1)],
        out_specs=[_row_spec(CH), _row_spec(HALF), _row_spec(HALF)],
        out_shape=[
            jax.ShapeDtypeStruct((N, CH), jnp.float32),
            jax.ShapeDtypeStruct((N, HALF), jnp.float32),
            jax.ShapeDtypeStruct((N, HALF), jnp.float32),
        ],
    )(*xs, *w_pieces, b.reshape(1, CH), wc, dinv)


def _branch_post(a0, a1, xw, dinv, inv_deg, bc):
    def body(a0_ref, a1_ref, xw_ref, d_ref, id_ref, b_ref, o_ref):
        acc = jnp.concatenate([a0_ref[...], a1_ref[...]], axis=1)
        o_ref[...] = jnp.maximum(
            d_ref[...] * acc + xw_ref[...] * id_ref[...] + b_ref[...], 0.0)

    return pl.pallas_call(
        body,
        grid=(NBLK,),
        in_specs=[
            _row_spec(HALF), _row_spec(HALF), _row_spec(CH),
            _row_spec(1), _row_spec(1), _rep_spec(1, CH),
        ],
        out_specs=_row_spec(CH),
        out_shape=jax.ShapeDtypeStruct((N, CH), jnp.float32),
    )(a0[:N], a1[:N], xw, dinv, inv_deg, bc.reshape(1, CH))


def _dinv_post(cnt_f, cnt_b):
    def body(cf_ref, cb_ref, df_ref, db_ref, idf_ref, idb_ref):
        deg_f = cf_ref[...] + 1.0
        deg_b = cb_ref[...] + 1.0
        df_ref[...] = lax.rsqrt(deg_f)
        db_ref[...] = lax.rsqrt(deg_b)
        idf_ref[...] = 1.0 / deg_f
        idb_ref[...] = 1.0 / deg_b

    return pl.pallas_call(
        body,
        grid=(NBLK,),
        in_specs=[_row_spec(1), _row_spec(1)],
        out_specs=[_row_spec(1)] * 4,
        out_shape=[jax.ShapeDtypeStruct((N, 1), jnp.float32)] * 4,
    )(cnt_f, cnt_b)


def _col_mean(xs):
    n = len(xs)

    def body(*refs):
        o_ref = refs[n]
        i = pl.program_id(0)

        @pl.when(i == 0)
        def _():
            o_ref[...] = jnp.zeros_like(o_ref)

        for k in range(n):
            o_ref[0:1, k * CH:(k + 1) * CH] += jnp.sum(
                refs[k][...], axis=0, keepdims=True) * (1.0 / N)

    return pl.pallas_call(
        body,
        grid=(NBLK,),
        in_specs=[_row_spec(CH)] * n,
        out_specs=pl.BlockSpec((1, n * CH), lambda i: (0, 0)),
        out_shape=jax.ShapeDtypeStruct((1, n * CH), jnp.float32),
    )(*xs)


def _head(hm, p, noise):
    def body(hm_ref, wh_ref, bh_ref, wm_ref, bm_ref, wv_ref, bv_ref,
             nz_ref, wdi_ref, bdi_ref, wdh_ref, bdh_ref,
             wi0_ref, bi0_ref, wci0_ref, wo0_ref, bo0_ref, wco0_ref,
             mean_ref, var_ref, d2_ref, xwi_ref, xwo_ref):
        h = jnp.maximum(
            jnp.dot(hm_ref[...], wh_ref[...],
                    preferred_element_type=jnp.float32) + bh_ref[...], 0.0)
        mean = 2.0 * jnp.tanh(
            jnp.dot(h, wm_ref[...], preferred_element_type=jnp.float32)
            + bm_ref[...])
        var = 2.0 * jax.nn.sigmoid(
            jnp.dot(h, wv_ref[...], preferred_element_type=jnp.float32)
            + bv_ref[...])
        mean_ref[...] = mean
        var_ref[...] = var
        z = mean + nz_ref[...] * jnp.sqrt(var)
        d = jnp.maximum(
            jnp.dot(z, wdi_ref[...], preferred_element_type=jnp.float32)
            + bdi_ref[...], 0.0)
        d = jnp.maximum(
            jnp.dot(d, wdh_ref[...], preferred_element_type=jnp.float32)
            + bdh_ref[...], 0.0)
        d2_ref[...] = d
        xp = jnp.maximum(
            jnp.dot(d, wi0_ref[...], preferred_element_type=jnp.float32)
            + bi0_ref[...], 0.0)
        xwi_ref[...] = jnp.dot(xp, wci0_ref[...],
                               preferred_element_type=jnp.float32)
        xp = jnp.maximum(
            jnp.dot(d, wo0_ref[...], preferred_element_type=jnp.float32)
            + bo0_ref[...], 0.0)
        xwo_ref[...] = jnp.dot(xp, wco0_ref[...],
                               preferred_element_type=jnp.float32)

    d0 = p["dec_dense"][0]
    ins = [hm,
           p["enc_hidden"]["W"], p["enc_hidden"]["b"].reshape(1, -1),
           p["enc_mean"]["W"], p["enc_mean"]["b"].reshape(1, -1),
           p["enc_var"]["W"], p["enc_var"]["b"].reshape(1, -1),
           noise,
           p["dec_input"]["W"], p["dec_input"]["b"].reshape(1, -1),
           p["dec_hidden"]["W"], p["dec_hidden"]["b"].reshape(1, -1),
           d0["lin_in"]["W"], d0["lin_in"]["b"].reshape(1, -1),
           d0["conv_in"]["W"],
           d0["lin_out"]["W"], d0["lin_out"]["b"].reshape(1, -1),
           d0["conv_out"]["W"]]
    return pl.pallas_call(
        body,
        grid=(1,),
        in_specs=[pl.BlockSpec(v.shape, lambda i: (0, 0)) for v in ins],
        out_specs=[pl.BlockSpec((1, CH), lambda i: (0, 0))] * 5,
        out_shape=[jax.ShapeDtypeStruct((1, CH), jnp.float32)] * 5,
    )(*ins)


def _dec0_outer(sf, sb, dinv_f, inv_deg_f, dinv_b, inv_deg_b,
                xwi, xwo, bi, bo):
    def body(sf_ref, sb_ref, df_ref, idf_ref, db_ref, idb_ref,
             xwi_ref, xwo_ref, bi_ref, bo_ref, xi_ref, xo_ref):
        cf = df_ref[...] * sf_ref[...] + idf_ref[...]
        xi_ref[...] = jnp.maximum(cf * xwi_ref[...] + bi_ref[...], 0.0)
        cb = db_ref[...] * sb_ref[...] + idb_ref[...]
        xo_ref[...] = jnp.maximum(cb * xwo_ref[...] + bo_ref[...], 0.0)

    return pl.pallas_call(
        body,
        grid=(NBLK,),
        in_specs=[_row_spec(1), _row_spec(1),
                  _row_spec(1), _row_spec(1), _row_spec(1), _row_spec(1),
                  _rep_spec(1, CH), _rep_spec(1, CH),
                  _rep_spec(1, CH), _rep_spec(1, CH)],
        out_specs=[_row_spec(CH), _row_spec(CH)],
        out_shape=[jax.ShapeDtypeStruct((N, CH), jnp.float32)] * 2,
    )(sf, sb, dinv_f, inv_deg_f, dinv_b, inv_deg_b,
      xwi, xwo, bi.reshape(1, CH), bo.reshape(1, CH))


def _out_proj(xs, w, b):
    n = len(xs)
    fo = w.shape[1]

    def body(*refs):
        wrefs = refs[n:2 * n]
        b_ref, o_ref = refs[2 * n:]
        a = jnp.broadcast_to(b_ref[...], o_ref.shape).astype(jnp.float32)
        for k in range(n):
            a = a + jnp.dot(refs[k][...], wrefs[k][...],
                            preferred_element_type=jnp.float32)
        o_ref[...] = a

    def piece_spec(v):
        return _rep_spec(1, CH) if v.shape[0] == 1 else _row_spec(CH)

    w_pieces = [w[k * CH:(k + 1) * CH] for k in range(n)]
    return pl.pallas_call(
        body,
        grid=(NBLK,),
        in_specs=[piece_spec(v) for v in xs] + [_rep_spec(CH, fo)] * n
        + [_rep_spec(1, fo)],
        out_specs=_row_spec(fo),
        out_shape=jax.ShapeDtypeStruct((N, fo), jnp.float32),
    )(*xs, *w_pieces, b.reshape(1, fo))


def _gcn_pair(xs, p, row_g, col_s, col_g, row_s,
              dinv_f, inv_deg_f, dinv_b, inv_deg_b):
    nin = len(xs)
    wi = [p["lin_in"]["W"][k * CH:(k + 1) * CH] for k in range(nin)]
    wo = [p["lin_out"]["W"][k * CH:(k + 1) * CH] for k in range(nin)]
    xwi, ui0, ui1 = _branch_pre(xs, wi, p["lin_in"]["b"],
                                p["conv_in"]["W"], dinv_f)
    xwo, uo0, uo1 = _branch_pre(xs, wo, p["lin_out"]["b"],
                                p["conv_out"]["W"], dinv_b)
    ai0, ai1 = _sc_edge_pass(ui0, ui1, row_g, col_s, row_g, col_s, HALF)
    ao0, ao1 = _sc_edge_pass(uo0, uo1, col_g, row_s, col_g, row_s, HALF)
    xi = _branch_post(ai0, ai1, xwi, dinv_f, inv_deg_f, p["conv_in"]["b"])
    xo = _branch_post(ao0, ao1, xwo, dinv_b, inv_deg_b, p["conv_out"]["b"])
    return xi, xo


def kernel(x, edge_index, params):
    row = edge_index[0]
    col = edge_index[1]
    padz = jnp.zeros((E_PAD - E,), jnp.int32)
    padt = jnp.full((E_PAD - E,), TRASH, jnp.int32)
    row_g = jnp.concatenate([row, padz])
    col_g = jnp.concatenate([col, padz])
    row_s = jnp.concatenate([row, padt])
    col_s = jnp.concatenate([col, padt])
    cnt_f, cnt_b = _sc_degrees(col_s, row_s)
    cf_n = cnt_f.reshape(N_SC)[:N].reshape(N, 1)
    cb_n = cnt_b.reshape(N_SC)[:N].reshape(N, 1)
    dinv_f, dinv_b, inv_deg_f, inv_deg_b = _dinv_post(cf_n, cb_n)
    zpad = jnp.zeros((N_SC - N,), jnp.float32)
    dtab_f = jnp.concatenate([dinv_f[:, 0], zpad]).reshape(HROWS, 16)
    dtab_b = jnp.concatenate([dinv_b[:, 0], zpad]).reshape(HROWS, 16)

    sf, sb = _sc_coef(dtab_f, dtab_b, row_g, col_s, col_g, row_s)
    sf = sf.reshape(N_SC)[:N].reshape(N, 1)
    sb = sb.reshape(N_SC)[:N].reshape(N, 1)

    h, xwi, ui0, ui1, xwo, uo0, uo1 = _embed_pre(
        x, params["embed"],
        params["enc_dense"][0]["lin_in"]["W"],
        params["enc_dense"][0]["lin_in"]["b"],
        params["enc_dense"][0]["conv_in"]["W"],
        params["enc_dense"][0]["lin_out"]["W"],
        params["enc_dense"][0]["lin_out"]["b"],
        params["enc_dense"][0]["conv_out"]["W"],
        dinv_f, dinv_b)
    ai0, ai1 = _sc_edge_pass(ui0, ui1, row_g, col_s, row_g, col_s, HALF)
    ao0, ao1 = _sc_edge_pass(uo0, uo1, col_g, row_s, col_g, row_s, HALF)
    xi = _branch_post(ai0, ai1, xwi, dinv_f, inv_deg_f,
                      params["enc_dense"][0]["conv_in"]["b"])
    xo = _branch_post(ao0, ao1, xwo, dinv_b, inv_deg_b,
                      params["enc_dense"][0]["conv_out"]["b"])
    xs = [h, xi, xo]
    xi2, xo2 = _gcn_pair(xs, params["enc_dense"][1], row_g, col_s, col_g,
                         row_s, dinv_f, inv_deg_f, dinv_b, inv_deg_b)
    xs = xs + [xi2, xo2]

    hm = _col_mean(xs)
    noise = jax.random.normal(jax.random.key(42), (1, CH), jnp.float32)
    mean, var, d2, xwi0, xwo0 = _head(hm, params, noise)

    d0 = params["dec_dense"][0]
    dxi, dxo = _dec0_outer(sf, sb, dinv_f, inv_deg_f, dinv_b, inv_deg_b,
                           xwi0, xwo0, d0["conv_in"]["b"],
                           d0["conv_out"]["b"])
    dxs = [d2, dxi, dxo]
    dxi2, dxo2 = _gcn_pair(dxs, params["dec_dense"][1], row_g, col_s, col_g,
                           row_s, dinv_f, inv_deg_f, dinv_b, inv_deg_b)
    dxs = dxs + [dxi2, dxo2]

    y = _out_proj(dxs, params["dec_output"]["W"], params["dec_output"]["b"])
    return (mean.reshape(CH), var.reshape(CH), y)


# speedup vs baseline: 1.1265x; 1.0020x over previous
import functools

import jax
import jax.numpy as jnp
from jax import lax
from jax.experimental import pallas as pl
from jax.experimental.pallas import tpu as pltpu
from jax.experimental.pallas import tpu_sc as plsc

N = 50000
E = 800000
CH = 64
HALF = 32
NS = 16
CHUNK = 128
TRASH = N
E_PAD = 819200
NCH_PT = E_PAD // (NS * CHUNK)
NG5 = NCH_PT
UNROLL = 8
DGB = 8
N_SC = 51200
ROWS_PT = N_SC // NS
ZROWS = 128
ROWS_BLK = 2000
NBLK = N // ROWS_BLK

@functools.cache
def _mesh():
    return plsc.VectorSubcoreMesh(core_axis_name="c", subcore_axis_name="s",
                                  num_cores=2, num_subcores=NS)
_SC_PARAMS = pltpu.CompilerParams(use_tc_tiling_on_sc=False)
_SC_PARAMS_REG = pltpu.CompilerParams(use_tc_tiling_on_sc=False,
                                      needs_layout_passes=False)


def _full16(v, dtype=jnp.float32):
    return jnp.full((16,), v, dtype)


HROWS = N_SC // 16
HSTRIPE = HROWS // NS
MROWS = 128
IDXB = 1024


def _reg_hist_zero(hist):
    @pl.loop(0, HROWS)
    def _(i):
        hist[i] = _full16(0.0)


def _reg_hist_merge(hist, iotav, acc, sem):
    @pl.loop(0, HROWS // MROWS)
    def _(m):
        @pl.loop(0, MROWS // 16)
        def _(i):
            iotav[pl.ds(i * 16, 16)] = (lax.iota(jnp.int32, 16)
                                        + (m * MROWS + i * 16))
        pltpu.async_copy(hist.at[pl.ds(m * MROWS, MROWS)],
                         acc.at[iotav], sem, add=True).wait()


def _deg_body_reg(cols_hbm, rows_hbm, degf_hbm, degb_hbm,
                  idxv, hist, iotav, acc, zbuf, sem_i, sem_m):
    c = lax.axis_index("c")
    s = lax.axis_index("s")
    ebase = s * (NCH_PT * CHUNK)
    EPT = NCH_PT * CHUNK

    _reg_hist_zero(hist)

    @pl.loop(0, HSTRIPE // 2)
    def _(i):
        zbuf[i] = _full16(0.0)

    r0 = s * HSTRIPE

    @pl.loop(0, 2)
    def _(zi):
        pltpu.sync_copy(zbuf, acc.at[pl.ds(r0 + zi * (HSTRIPE // 2),
                                           HSTRIPE // 2)])

    ones = _full16(1.0)

    def load(slot, b):
        @pl.when(c == 0)
        def _():
            pltpu.async_copy(cols_hbm.at[pl.ds(ebase + b * IDXB, IDXB)],
                             idxv.at[slot], sem_i)

        @pl.when(c == 1)
        def _():
            pltpu.async_copy(rows_hbm.at[pl.ds(ebase + b * IDXB, IDXB)],
                             idxv.at[slot], sem_i)

    def wait(slot):
        pltpu.make_async_copy(cols_hbm.at[pl.ds(0, IDXB)],
                              idxv.at[slot], sem_i).wait()

    def consume(slot):
        @pl.loop(0, IDXB // 16)
        def _(i):
            v = idxv[slot, pl.ds(i * 16, 16)]
            r = jax.lax.shift_right_logical(v, 4)
            col = jax.lax.bitwise_and(v, 15)
            plsc.addupdate_scatter(hist, [r, col], ones)

    NB = EPT // IDXB
    load(0, 0)

    @pl.loop(0, NB // 2)
    def _(b):
        wait(0)

        @pl.when(2 * b + 1 < NB)
        def _():
            load(1, 2 * b + 1)
        consume(0)

        wait(1)

        @pl.when(2 * b + 2 < NB)
        def _():
            load(0, 2 * b + 2)
        consume(1)

    plsc.subcore_barrier()
    _reg_hist_merge(hist, iotav, acc, sem_m)
    plsc.subcore_barrier()

    @pl.when(c == 0)
    def _():
        pltpu.sync_copy(acc.at[pl.ds(r0, HSTRIPE)],
                        degf_hbm.at[pl.ds(r0, HSTRIPE)])

    @pl.when(c == 1)
    def _():
        pltpu.sync_copy(acc.at[pl.ds(r0, HSTRIPE)],
                        degb_hbm.at[pl.ds(r0, HSTRIPE)])


def _sc_degrees(col_s, row_s):
    f = pl.kernel(
        _deg_body_reg,
        out_type=[
            jax.ShapeDtypeStruct((HROWS, 16), jnp.float32),
            jax.ShapeDtypeStruct((HROWS, 16), jnp.float32),
        ],
        mesh=_mesh(),
        scratch_types=[
            pltpu.VMEM((2, IDXB), jnp.int32),
            pltpu.VMEM((HROWS, 16), jnp.float32),
            pltpu.VMEM((MROWS,), jnp.int32),
            pltpu.VMEM_SHARED((HROWS, 16), jnp.float32),
            pltpu.VMEM((HSTRIPE // 2, 16), jnp.float32),
            pltpu.SemaphoreType.DMA,
            pltpu.SemaphoreType.DMA,
        ],
        compiler_params=_SC_PARAMS_REG,
    )
    return f(col_s, row_s)


def _coef_body_reg(tf_hbm, tb_hbm, g0_hbm, s0_hbm, g1_hbm, s1_hbm,
                   sf_hbm, sb_hbm, gidxv, sidxv, dtab, hist, iotav, acc,
                   zbuf, sem_i, sem_m):
    c = lax.axis_index("c")
    s = lax.axis_index("s")
    ebase = s * (NCH_PT * CHUNK)
    EPT = NCH_PT * CHUNK

    _reg_hist_zero(hist)

    @pl.loop(0, HSTRIPE // 2)
    def _(i):
        zbuf[i] = _full16(0.0)

    r0 = s * HSTRIPE

    @pl.loop(0, 2)
    def _(zi):
        pltpu.sync_copy(zbuf, acc.at[pl.ds(r0 + zi * (HSTRIPE // 2),
                                           HSTRIPE // 2)])

    @pl.when(c == 0)
    def _():
        pltpu.sync_copy(tf_hbm, dtab)

    @pl.when(c == 1)
    def _():
        pltpu.sync_copy(tb_hbm, dtab)

    def load(slot, b):
        @pl.when(c == 0)
        def _():
            pltpu.async_copy(g0_hbm.at[pl.ds(ebase + b * IDXB, IDXB)],
                             gidxv.at[slot], sem_i)
            pltpu.async_copy(s0_hbm.at[pl.ds(ebase + b * IDXB, IDXB)],
                             sidxv.at[slot], sem_i)

        @pl.when(c == 1)
        def _():
            pltpu.async_copy(g1_hbm.at[pl.ds(ebase + b * IDXB, IDXB)],
                             gidxv.at[slot], sem_i)
            pltpu.async_copy(s1_hbm.at[pl.ds(ebase + b * IDXB, IDXB)],
                             sidxv.at[slot], sem_i)

    def wait(slot):
        pltpu.make_async_copy(g0_hbm.at[pl.ds(0, IDXB)],
                              gidxv.at[slot], sem_i).wait()
        pltpu.make_async_copy(g0_hbm.at[pl.ds(0, IDXB)],
                              sidxv.at[slot], sem_i).wait()

    def consume(slot):
        @pl.loop(0, IDXB // 16)
        def _(i):
            gv = gidxv[slot, pl.ds(i * 16, 16)]
            gr = jax.lax.shift_right_logical(gv, 4)
            gc = jax.lax.bitwise_and(gv, 15)
            vals = plsc.load_gather(dtab, [gr, gc])
            sv = sidxv[slot, pl.ds(i * 16, 16)]
            sr = jax.lax.shift_right_logical(sv, 4)
            sc_ = jax.lax.bitwise_and(sv, 15)
            plsc.addupdate_scatter(hist, [sr, sc_], vals)

    NB = EPT // IDXB
    load(0, 0)

    @pl.loop(0, NB // 2)
    def _(b):
        wait(0)

        @pl.when(2 * b + 1 < NB)
        def _():
            load(1, 2 * b + 1)
        consume(0)

        wait(1)

        @pl.when(2 * b + 2 < NB)
        def _():
            load(0, 2 * b + 2)
        consume(1)

    plsc.subcore_barrier()
    _reg_hist_merge(hist, iotav, acc, sem_m)
    plsc.subcore_barrier()

    @pl.when(c == 0)
    def _():
        pltpu.sync_copy(acc.at[pl.ds(r0, HSTRIPE)],
                        sf_hbm.at[pl.ds(r0, HSTRIPE)])

    @pl.when(c == 1)
    def _():
        pltpu.sync_copy(acc.at[pl.ds(r0, HSTRIPE)],
                        sb_hbm.at[pl.ds(r0, HSTRIPE)])


def _sc_coef(tab_f, tab_b, row_g, col_s, col_g, row_s):
    f = pl.kernel(
        _coef_body_reg,
        out_type=[
            jax.ShapeDtypeStruct((HROWS, 16), jnp.float32),
            jax.ShapeDtypeStruct((HROWS, 16), jnp.float32),
        ],
        mesh=_mesh(),
        scratch_types=[
            pltpu.VMEM((2, IDXB), jnp.int32),
            pltpu.VMEM((2, IDXB), jnp.int32),
            pltpu.VMEM((HROWS, 16), jnp.float32),
            pltpu.VMEM((HROWS, 16), jnp.float32),
            pltpu.VMEM((MROWS,), jnp.int32),
            pltpu.VMEM_SHARED((HROWS, 16), jnp.float32),
            pltpu.VMEM((HSTRIPE // 2, 16), jnp.float32),
            pltpu.SemaphoreType.DMA,
            pltpu.SemaphoreType.DMA,
        ],
        compiler_params=_SC_PARAMS_REG,
    )
    return f(tab_f, tab_b, row_g, col_s, col_g, row_s)


def _make_edge_body_v5(width):
    def body(u0_hbm, u1_hbm, g0_hbm, s0_hbm, g1_hbm, s1_hbm,
             out0_hbm, out1_hbm, rowv, colv, buf, zbuf, acc,
             sem_i0, sem_i1, sem_g0, sem_g1, sem_g2, sem_g3,
             sem_s0, sem_s1, sem_s2, sem_s3):
        sem_i = [sem_i0, sem_i1]
        sem_g = [sem_g0, sem_g1, sem_g2, sem_g3]
        sem_s = [sem_s0, sem_s1, sem_s2, sem_s3]
        c = lax.axis_index("c")
        s = lax.axis_index("s")
        tbase = s * (NCH_PT * CHUNK)

        @pl.loop(0, ZROWS)
        def _(i):
            for w in range(width // 16):
                zbuf[i, pl.ds(w * 16, 16)] = _full16(0.0)

        r0 = s * ROWS_PT

        @pl.loop(0, ROWS_PT // ZROWS)
        def _(zi):
            pltpu.sync_copy(zbuf, acc.at[pl.ds(r0 + zi * ZROWS, ZROWS)])

        plsc.subcore_barrier()

        def idx_start(p, g):
            base = tbase + g * CHUNK
            s8 = p % 8
            sem = sem_i[p % 2]

            @pl.when(c == 0)
            def _():
                pltpu.async_copy(g0_hbm.at[pl.ds(base, CHUNK)],
                                 rowv.at[s8], sem)
                pltpu.async_copy(s0_hbm.at[pl.ds(base, CHUNK)],
                                 colv.at[s8], sem)

            @pl.when(c == 1)
            def _():
                pltpu.async_copy(g1_hbm.at[pl.ds(base, CHUNK)],
                                 rowv.at[s8], sem)
                pltpu.async_copy(s1_hbm.at[pl.ds(base, CHUNK)],
                                 colv.at[s8], sem)

        def idx_wait(p):
            s8 = p % 8
            sem = sem_i[p % 2]
            pltpu.make_async_copy(g0_hbm.at[pl.ds(0, CHUNK)],
                                  rowv.at[s8], sem).wait()
            pltpu.make_async_copy(g0_hbm.at[pl.ds(0, CHUNK)],
                                  colv.at[s8], sem).wait()

        def gather_start(p):
            s4, s8 = p % 4, p % 8

            @pl.when(c == 0)
            def _():
                pltpu.async_copy(u0_hbm.at[rowv.at[s8]], buf.at[s4],
                                 sem_g[s4])

            @pl.when(c == 1)
            def _():
                pltpu.async_copy(u1_hbm.at[rowv.at[s8]], buf.at[s4],
                                 sem_g[s4])

        def gather_wait(p):
            s4, s8 = p % 4, p % 8
            pltpu.make_async_copy(u0_hbm.at[rowv.at[s8]], buf.at[s4],
                                  sem_g[s4]).wait()

        def scatter_start(p):
            s4, s8 = p % 4, p % 8
            pltpu.async_copy(buf.at[s4], acc.at[colv.at[s8]], sem_s[s4],
                             add=True)

        def scatter_wait(p):
            s4, s8 = p % 4, p % 8
            pltpu.make_async_copy(buf.at[s4], acc.at[colv.at[s8]],
                                  sem_s[s4]).wait()

        idx_start(0, 0)
        idx_start(1, 1)

        @pl.loop(0, NG5 // UNROLL)
        def _(k):
            for p in range(UNROLL):
                g = k * UNROLL + p

                if p >= 4:
                    scatter_wait(p - 4)
                else:
                    @pl.when(k > 0)
                    def _():
                        scatter_wait(p + 4)

                idx_wait(p)

                gather_start(p)

                if p >= 3:
                    gather_wait(p - 3)
                    scatter_start(p - 3)
                else:
                    @pl.when(k > 0)
                    def _():
                        gather_wait(p + 5)
                        scatter_start(p + 5)

                if p < 6:
                    idx_start(p + 2, g + 2)
                else:
                    @pl.when(k + 1 < NG5 // UNROLL)
                    def _():
                        idx_start(p + 2, g + 2)

        for p, g in ((5, NG5 - 3), (6, NG5 - 2), (7, NG5 - 1)):
            gather_wait(p)
            scatter_start(p)
        for p in (4, 5, 6, 7):
            scatter_wait(p)

        plsc.subcore_barrier()

        @pl.when(c == 0)
        def _():
            pltpu.sync_copy(acc.at[pl.ds(r0, ROWS_PT)],
                            out0_hbm.at[pl.ds(r0, ROWS_PT)])

        @pl.when(c == 1)
        def _():
            pltpu.sync_copy(acc.at[pl.ds(r0, ROWS_PT)],
                            out1_hbm.at[pl.ds(r0, ROWS_PT)])

    return body


def _sc_edge_pass(u0, u1, g0, s0, g1, s1, width):
    f = pl.kernel(
        _make_edge_body_v5(width),
        out_type=[
            jax.ShapeDtypeStruct((N_SC, width), jnp.float32),
            jax.ShapeDtypeStruct((N_SC, width), jnp.float32),
        ],
        mesh=_mesh(),
        scratch_types=[
            pltpu.VMEM((8, CHUNK), jnp.int32),
            pltpu.VMEM((8, CHUNK), jnp.int32),
            pltpu.VMEM((4, CHUNK, width), jnp.float32),
            pltpu.VMEM((ZROWS, width), jnp.float32),
            pltpu.VMEM_SHARED((N_SC, width), jnp.float32),
        ] + [pltpu.SemaphoreType.DMA] * 10,
        compiler_params=_SC_PARAMS,
    )
    return f(u0, u1, g0, s0, g1, s1)


def _row_spec(cols):
    return pl.BlockSpec((ROWS_BLK, cols), lambda i: (i, 0))


def _rep_spec(r, cols):
    return pl.BlockSpec((r, cols), lambda i: (0, 0))


def _embed_pre(x, embed, wi, bi, wci, wo, bo, wco, dinv_f, dinv_b):
    def body(x_ref, emb_ref, wi_ref, bi_ref, wci_ref, wo_ref, bo_ref,
             wco_ref, df_ref, db_ref, h_ref,
             xwi_ref, ui0_ref, ui1_ref, xwo_ref, uo0_ref, uo1_ref):
        ids = x_ref[...]
        onehot = (ids == lax.broadcasted_iota(jnp.int32, (ROWS_BLK, 32), 1)
                  ).astype(jnp.float32)
        h = jnp.dot(onehot, emb_ref[...], preferred_element_type=jnp.float32)
        h_ref[...] = h
        xp = jnp.maximum(
            jnp.dot(h, wi_ref[...], preferred_element_type=jnp.float32)
            + bi_ref[...], 0.0)
        xw = jnp.dot(xp, wci_ref[...], preferred_element_type=jnp.float32)
        xwi_ref[...] = xw
        u = xw * df_ref[...]
        ui0_ref[...] = u[:, :HALF]
        ui1_ref[...] = u[:, HALF:]
        xp = jnp.maximum(
            jnp.dot(h, wo_ref[...], preferred_element_type=jnp.float32)
            + bo_ref[...], 0.0)
        xw = jnp.dot(xp, wco_ref[...], preferred_element_type=jnp.float32)
        xwo_ref[...] = xw
        u = xw * db_ref[...]
        uo0_ref[...] = u[:, :HALF]
        uo1_ref[...] = u[:, HALF:]

    return pl.pallas_call(
        body,
        grid=(NBLK,),
        in_specs=[
            _row_spec(1),
            _rep_spec(32, CH),
            _rep_spec(CH, CH), _rep_spec(1, CH), _rep_spec(CH, CH),
            _rep_spec(CH, CH), _rep_spec(1, CH), _rep_spec(CH, CH),
            _row_spec(1), _row_spec(1),
        ],
        out_specs=[
            _row_spec(CH),
            _row_spec(CH), _row_spec(HALF), _row_spec(HALF),
            _row_spec(CH), _row_spec(HALF), _row_spec(HALF),
        ],
        out_shape=[
            jax.ShapeDtypeStruct((N, CH), jnp.float32),
            jax.ShapeDtypeStruct((N, CH), jnp.float32),
            jax.ShapeDtypeStruct((N, HALF), jnp.float32),
            jax.ShapeDtypeStruct((N, HALF), jnp.float32),
            jax.ShapeDtypeStruct((N, CH), jnp.float32),
            jax.ShapeDtypeStruct((N, HALF), jnp.float32),
            jax.ShapeDtypeStruct((N, HALF), jnp.float32),
        ],
    )(x.reshape(N, 1), embed, wi, bi.reshape(1, CH), wci,
      wo, bo.reshape(1, CH), wco, dinv_f, dinv_b)


def _piece_spec(v):
    return _rep_spec(1, CH) if v.shape[0] == 1 else _row_spec(CH)


def _branch_pre(xs, w_pieces, b, wc, dinv):
    n = len(xs)

    def body(*refs):
        xrefs = refs[:n]
        wrefs = refs[n:2 * n]
        b_ref, wc_ref, d_ref, xw_ref, u0_ref, u1_ref = refs[2 * n:]
        a = b_ref[...].astype(jnp.float32)
        for k in range(n):
            a = a + jnp.dot(xrefs[k][...], wrefs[k][...],
                            preferred_element_type=jnp.float32)
        xp = jnp.maximum(a, 0.0)
        xw = jnp.dot(xp, wc_ref[...], preferred_element_type=jnp.float32)
        xw_ref[...] = xw
        u = xw * d_ref[...]
        u0_ref[...] = u[:, :HALF]
        u1_ref[...] = u[:, HALF:]

    return pl.pallas_call(
        body,
        grid=(NBLK,),
        in_specs=[_piece_spec(v) for v in xs] + [_rep_spec(CH, CH)] * n
        + [_rep_spec(1, CH), _rep_spec(CH, CH), _row_spec(1)],
        out_specs=[_row_spec(CH), _row_spec(HALF), _row_spec(HALF)],
        out_shape=[
            jax.ShapeDtypeStruct((N, CH), jnp.float32),
            jax.ShapeDtypeStruct((N, HALF), jnp.float32),
            jax.ShapeDtypeStruct((N, HALF), jnp.float32),
        ],
    )(*xs, *w_pieces, b.reshape(1, CH), wc, dinv)


def _branch_post(a0, a1, xw, dinv, inv_deg, bc):
    def body(a0_ref, a1_ref, xw_ref, d_ref, id_ref, b_ref, o_ref):
        acc = jnp.concatenate([a0_ref[...], a1_ref[...]], axis=1)
        o_ref[...] = jnp.maximum(
            d_ref[...] * acc + xw_ref[...] * id_ref[...] + b_ref[...], 0.0)

    return pl.pallas_call(
        body,
        grid=(NBLK,),
        in_specs=[
            _row_spec(HALF), _row_spec(HALF), _row_spec(CH),
            _row_spec(1), _row_spec(1), _rep_spec(1, CH),
        ],
        out_specs=_row_spec(CH),
        out_shape=jax.ShapeDtypeStruct((N, CH), jnp.float32),
    )(a0[:N], a1[:N], xw, dinv, inv_deg, bc.reshape(1, CH))


def _dinv_post(cnt_f, cnt_b):
    def body(cf_ref, cb_ref, df_ref, db_ref, idf_ref, idb_ref):
        deg_f = cf_ref[...] + 1.0
        deg_b = cb_ref[...] + 1.0
        df_ref[...] = lax.rsqrt(deg_f)
        db_ref[...] = lax.rsqrt(deg_b)
        idf_ref[...] = 1.0 / deg_f
        idb_ref[...] = 1.0 / deg_b

    return pl.pallas_call(
        body,
        grid=(NBLK,),
        in_specs=[_row_spec(1), _row_spec(1)],
        out_specs=[_row_spec(1)] * 4,
        out_shape=[jax.ShapeDtypeStruct((N, 1), jnp.float32)] * 4,
    )(cnt_f, cnt_b)


def _col_mean(xs):
    n = len(xs)

    def body(*refs):
        o_ref = refs[n]
        i = pl.program_id(0)

        @pl.when(i == 0)
        def _():
            o_ref[...] = jnp.zeros_like(o_ref)

        for k in range(n):
            o_ref[0:1, k * CH:(k + 1) * CH] += jnp.sum(
                refs[k][...], axis=0, keepdims=True) * (1.0 / N)

    return pl.pallas_call(
        body,
        grid=(NBLK,),
        in_specs=[_row_spec(CH)] * n,
        out_specs=pl.BlockSpec((1, n * CH), lambda i: (0, 0)),
        out_shape=jax.ShapeDtypeStruct((1, n * CH), jnp.float32),
    )(*xs)


def _head(hm, p, noise):
    def body(hm_ref, wh_ref, bh_ref, wm_ref, bm_ref, wv_ref, bv_ref,
             nz_ref, wdi_ref, bdi_ref, wdh_ref, bdh_ref,
             wi0_ref, bi0_ref, wci0_ref, wo0_ref, bo0_ref, wco0_ref,
             mean_ref, var_ref, d2_ref, xwi_ref, xwo_ref):
        h = jnp.maximum(
            jnp.dot(hm_ref[...], wh_ref[...],
                    preferred_element_type=jnp.float32) + bh_ref[...], 0.0)
        mean = 2.0 * jnp.tanh(
            jnp.dot(h, wm_ref[...], preferred_element_type=jnp.float32)
            + bm_ref[...])
        var = 2.0 * jax.nn.sigmoid(
            jnp.dot(h, wv_ref[...], preferred_element_type=jnp.float32)
            + bv_ref[...])
        mean_ref[...] = mean
        var_ref[...] = var
        z = mean + nz_ref[...] * jnp.sqrt(var)
        d = jnp.maximum(
            jnp.dot(z, wdi_ref[...], preferred_element_type=jnp.float32)
            + bdi_ref[...], 0.0)
        d = jnp.maximum(
            jnp.dot(d, wdh_ref[...], preferred_element_type=jnp.float32)
            + bdh_ref[...], 0.0)
        d2_ref[...] = d
        xp = jnp.maximum(
            jnp.dot(d, wi0_ref[...], preferred_element_type=jnp.float32)
            + bi0_ref[...], 0.0)
        xwi_ref[...] = jnp.dot(xp, wci0_ref[...],
                               preferred_element_type=jnp.float32)
        xp = jnp.maximum(
            jnp.dot(d, wo0_ref[...], preferred_element_type=jnp.float32)
            + bo0_ref[...], 0.0)
        xwo_ref[...] = jnp.dot(xp, wco0_ref[...],
                               preferred_element_type=jnp.float32)

    d0 = p["dec_dense"][0]
    ins = [hm,
           p["enc_hidden"]["W"], p["enc_hidden"]["b"].reshape(1, -1),
           p["enc_mean"]["W"], p["enc_mean"]["b"].reshape(1, -1),
           p["enc_var"]["W"], p["enc_var"]["b"].reshape(1, -1),
           noise,
           p["dec_input"]["W"], p["dec_input"]["b"].reshape(1, -1),
           p["dec_hidden"]["W"], p["dec_hidden"]["b"].reshape(1, -1),
           d0["lin_in"]["W"], d0["lin_in"]["b"].reshape(1, -1),
           d0["conv_in"]["W"],
           d0["lin_out"]["W"], d0["lin_out"]["b"].reshape(1, -1),
           d0["conv_out"]["W"]]
    return pl.pallas_call(
        body,
        grid=(1,),
        in_specs=[pl.BlockSpec(v.shape, lambda i: (0, 0)) for v in ins],
        out_specs=[pl.BlockSpec((1, CH), lambda i: (0, 0))] * 5,
        out_shape=[jax.ShapeDtypeStruct((1, CH), jnp.float32)] * 5,
    )(*ins)


def _dec0_outer(sf, sb, dinv_f, inv_deg_f, dinv_b, inv_deg_b,
                xwi, xwo, bi, bo):
    def body(sf_ref, sb_ref, df_ref, idf_ref, db_ref, idb_ref,
             xwi_ref, xwo_ref, bi_ref, bo_ref, xi_ref, xo_ref):
        cf = df_ref[...] * sf_ref[...] + idf_ref[...]
        xi_ref[...] = jnp.maximum(cf * xwi_ref[...] + bi_ref[...], 0.0)
        cb = db_ref[...] * sb_ref[...] + idb_ref[...]
        xo_ref[...] = jnp.maximum(cb * xwo_ref[...] + bo_ref[...], 0.0)

    return pl.pallas_call(
        body,
        grid=(NBLK,),
        in_specs=[_row_spec(1), _row_spec(1),
                  _row_spec(1), _row_spec(1), _row_spec(1), _row_spec(1),
                  _rep_spec(1, CH), _rep_spec(1, CH),
                  _rep_spec(1, CH), _rep_spec(1, CH)],
        out_specs=[_row_spec(CH), _row_spec(CH)],
        out_shape=[jax.ShapeDtypeStruct((N, CH), jnp.float32)] * 2,
    )(sf, sb, dinv_f, inv_deg_f, dinv_b, inv_deg_b,
      xwi, xwo, bi.reshape(1, CH), bo.reshape(1, CH))


def _out_proj(xs, w, b):
    n = len(xs)
    fo = w.shape[1]

    def body(*refs):
        wrefs = refs[n:2 * n]
        b_ref, o_ref = refs[2 * n:]
        a = jnp.broadcast_to(b_ref[...], o_ref.shape).astype(jnp.float32)
        for k in range(n):
            a = a + jnp.dot(refs[k][...], wrefs[k][...],
                            preferred_element_type=jnp.float32)
        o_ref[...] = a

    def piece_spec(v):
        return _rep_spec(1, CH) if v.shape[0] == 1 else _row_spec(CH)

    w_pieces = [w[k * CH:(k + 1) * CH] for k in range(n)]
    return pl.pallas_call(
        body,
        grid=(NBLK,),
        in_specs=[piece_spec(v) for v in xs] + [_rep_spec(CH, fo)] * n
        + [_rep_spec(1, fo)],
        out_specs=_row_spec(fo),
        out_shape=jax.ShapeDtypeStruct((N, fo), jnp.float32),
    )(*xs, *w_pieces, b.reshape(1, fo))


def _gcn_pair(xs, p, row_g, col_s, col_g, row_s,
              dinv_f, inv_deg_f, dinv_b, inv_deg_b):
    nin = len(xs)
    wi = [p["lin_in"]["W"][k * CH:(k + 1) * CH] for k in range(nin)]
    wo = [p["lin_out"]["W"][k * CH:(k + 1) * CH] for k in range(nin)]
    xwi, ui0, ui1 = _branch_pre(xs, wi, p["lin_in"]["b"],
                                p["conv_in"]["W"], dinv_f)
    xwo, uo0, uo1 = _branch_pre(xs, wo, p["lin_out"]["b"],
                                p["conv_out"]["W"], dinv_b)
    ai0, ai1 = _sc_edge_pass(ui0, ui1, row_g, col_s, row_g, col_s, HALF)
    ao0, ao1 = _sc_edge_pass(uo0, uo1, col_g, row_s, col_g, row_s, HALF)
    xi = _branch_post(ai0, ai1, xwi, dinv_f, inv_deg_f, p["conv_in"]["b"])
    xo = _branch_post(ao0, ao1, xwo, dinv_b, inv_deg_b, p["conv_out"]["b"])
    return xi, xo


def kernel(x, edge_index, params):
    row = edge_index[0]
    col = edge_index[1]
    padz = jnp.zeros((E_PAD - E,), jnp.int32)
    padt = jnp.full((E_PAD - E,), TRASH, jnp.int32)
    row_g = jnp.concatenate([row, padz])
    col_g = jnp.concatenate([col, padz])
    row_s = jnp.concatenate([row, padt])
    col_s = jnp.concatenate([col, padt])
    cnt_f, cnt_b = _sc_degrees(col_s, row_s)
    cf_n = cnt_f.reshape(N_SC)[:N].reshape(N, 1)
    cb_n = cnt_b.reshape(N_SC)[:N].reshape(N, 1)
    dinv_f, dinv_b, inv_deg_f, inv_deg_b = _dinv_post(cf_n, cb_n)
    zpad = jnp.zeros((N_SC - N,), jnp.float32)
    dtab_f = jnp.concatenate([dinv_f[:, 0], zpad]).reshape(HROWS, 16)
    dtab_b = jnp.concatenate([dinv_b[:, 0], zpad]).reshape(HROWS, 16)

    sf, sb = _sc_coef(dtab_f, dtab_b, row_g, col_s, col_g, row_s)
    sf = sf.reshape(N_SC)[:N].reshape(N, 1)
    sb = sb.reshape(N_SC)[:N].reshape(N, 1)

    h, xwi, ui0, ui1, xwo, uo0, uo1 = _embed_pre(
        x, params["embed"],
        params["enc_dense"][0]["lin_in"]["W"],
        params["enc_dense"][0]["lin_in"]["b"],
        params["enc_dense"][0]["conv_in"]["W"],
        params["enc_dense"][0]["lin_out"]["W"],
        params["enc_dense"][0]["lin_out"]["b"],
        params["enc_dense"][0]["conv_out"]["W"],
        dinv_f, dinv_b)
    ai0, ai1 = _sc_edge_pass(ui0, ui1, row_g, col_s, row_g, col_s, HALF)
    ao0, ao1 = _sc_edge_pass(uo0, uo1, col_g, row_s, col_g, row_s, HALF)
    xi = _branch_post(ai0, ai1, xwi, dinv_f, inv_deg_f,
                      params["enc_dense"][0]["conv_in"]["b"])
    xo = _branch_post(ao0, ao1, xwo, dinv_b, inv_deg_b,
                      params["enc_dense"][0]["conv_out"]["b"])
    xs = [h, xi, xo]
    xi2, xo2 = _gcn_pair(xs, params["enc_dense"][1], row_g, col_s, col_g,
                         row_s, dinv_f, inv_deg_f, dinv_b, inv_deg_b)
    xs = xs + [xi2, xo2]

    hm = _col_mean(xs)
    noise = jax.random.normal(jax.random.key(42), (1, CH), jnp.float32)
    mean, var, d2, xwi0, xwo0 = _head(hm, params, noise)

    d0 = params["dec_dense"][0]
    dxi, dxo = _dec0_outer(sf, sb, dinv_f, inv_deg_f, dinv_b, inv_deg_b,
                           xwi0, xwo0, d0["conv_in"]["b"],
                           d0["conv_out"]["b"])
    dxs = [d2, dxi, dxo]
    dxi2, dxo2 = _gcn_pair(dxs, params["dec_dense"][1], row_g, col_s, col_g,
                           row_s, dinv_f, inv_deg_f, dinv_b, inv_deg_b)
    dxs = dxs + [dxi2, dxo2]

    y = _out_proj(dxs, params["dec_output"]["W"], params["dec_output"]["b"])
    return (mean.reshape(CH), var.reshape(CH), y)
